# Initial kernel scaffold; baseline (speedup 1.0000x reference)
import jax
import jax.numpy as jnp
from jax.experimental import pallas as pl


def kernel(symbol_emb, gcn_w, gcn_wb, gcn_b, gate_w, gate_wb, gate_b, p1w, p1b, p2w, p2b, lng, lnb, wih, whh, bih, bhh, query, support, q_l1, q_e2, q_deg_l, q_r1, q_e5, q_deg_r, s_l1, s_e2, s_deg_l, s_r1, s_e5, s_deg_r):
    raise NotImplementedError("write your pallas kernel here")


# speedup vs baseline: 1.4110x; 1.4110x over previous
import functools

import jax
import jax.numpy as jnp
from jax import lax
from jax.experimental import pallas as pl
from jax.experimental.pallas import tpu as pltpu
from jax.experimental.pallas import tpu_sc as plsc

EMBED_DIM = 128
NUM_SYM = 100000
PAD = NUM_SYM
KMAX = 200
KSEL = 10
B_Q, B_S = 1024, 5
NROWS = 2 * B_Q + 2 * B_S
NB = 2304
NW = 32
CHUNK = 128


def _mm_t(a, b):
    return lax.dot_general(a, b, (((1,), (1,)), ((), ())),
                           preferred_element_type=jnp.float32)


def _sc_gather(table, ids_2d, nch):
    n_out = NW * nch * CHUNK
    per_w = nch * CHUNK
    mesh = plsc.VectorSubcoreMesh(core_axis_name="c", subcore_axis_name="s")

    @functools.partial(
        pl.kernel, mesh=mesh,
        out_type=jax.ShapeDtypeStruct((n_out, EMBED_DIM), jnp.float32),
        scratch_types=[
            pltpu.VMEM((nch, CHUNK), jnp.int32),
            pltpu.VMEM((CHUNK, EMBED_DIM), jnp.float32),
            pltpu.VMEM((CHUNK, EMBED_DIM), jnp.float32),
            pltpu.SemaphoreType.DMA,
            pltpu.SemaphoreType.DMA,
        ],
    )
    def k(table_hbm, ids_hbm, out_hbm, idx_v, buf0, buf1, sem0, sem1):
        wid = lax.axis_index("s") * 2 + lax.axis_index("c")
        pltpu.sync_copy(ids_hbm.at[wid], idx_v)
        bufs = (buf0, buf1)
        sems = (sem0, sem1)

        def start(ci, b):
            pltpu.make_async_copy(
                table_hbm.at[idx_v.at[ci]], bufs[b], sems[b]).start()

        def drain(ci, b):
            pltpu.make_async_copy(
                table_hbm.at[idx_v.at[ci]], bufs[b], sems[b]).wait()
            off = wid * per_w + ci * CHUNK
            pltpu.sync_copy(bufs[b], out_hbm.at[pl.ds(off, CHUNK)])

        start(0, 0)

        def body(i, carry):
            ci0 = 2 * i
            ci1 = ci0 + 1

            @pl.when(ci1 < nch)
            def _():
                start(ci1, 1)

            drain(ci0, 0)

            @pl.when(ci0 + 2 < nch)
            def _():
                start(ci0 + 2, 0)

            @pl.when(ci1 < nch)
            def _():
                drain(ci1, 1)

            return carry

        lax.fori_loop(0, (nch + 1) // 2, body, 0)

    return k(table, ids_2d)


def _tc_simtopk(rows_a, rel_ids, ent_ids, blk):
    grid = NB // blk
    self_off = NB * KMAX // blk

    def body(ent_ref, self_ref, rel_ref, eid_ref, relo_ref, ento_ref):
        ent = ent_ref[...].reshape(blk, KMAX, EMBED_DIM)
        selfr = self_ref[...]
        rels = rel_ref[...]
        eids = eid_ref[...]

        sn = selfr * lax.rsqrt(jnp.maximum(
            jnp.sum(selfr * selfr, axis=-1, keepdims=True), 1e-24))
        en = ent * lax.rsqrt(jnp.maximum(
            jnp.sum(ent * ent, axis=-1, keepdims=True), 1e-24))
        sim = jnp.sum(sn[:, None, :] * en, axis=-1)
        sim = sim - jnp.where(rels == PAD, 1e9, 0.0).astype(jnp.float32)

        iota = lax.broadcasted_iota(jnp.int32, (blk, KMAX), 1)
        relf = rels.astype(jnp.float32)
        entf = eids.astype(jnp.float32)
        rel_cols, ent_cols = [], []
        for _ in range(KSEL):
            m = jnp.max(sim, axis=1, keepdims=True)
            ismax = sim == m
            idxsel = jnp.min(jnp.where(ismax, iota, KMAX + 1), axis=1,
                             keepdims=True)
            chosen = iota == idxsel
            rel_cols.append(jnp.sum(jnp.where(chosen, relf, 0.0), axis=1,
                                    keepdims=True))
            ent_cols.append(jnp.sum(jnp.where(chosen, entf, 0.0), axis=1,
                                    keepdims=True))
            sim = jnp.where(chosen, -1e38, sim)
        relo_ref[...] = jnp.concatenate(rel_cols, axis=1).astype(jnp.int32)
        ento_ref[...] = jnp.concatenate(ent_cols, axis=1).astype(jnp.int32)

    return pl.pallas_call(
        body,
        grid=(grid,),
        in_specs=[
            pl.BlockSpec((blk * KMAX, EMBED_DIM), lambda g: (g, 0)),
            pl.BlockSpec((blk, EMBED_DIM), lambda g: (self_off + g, 0)),
            pl.BlockSpec((blk, KMAX), lambda g: (g, 0)),
            pl.BlockSpec((blk, KMAX), lambda g: (g, 0)),
        ],
        out_specs=[
            pl.BlockSpec((blk, KSEL), lambda g: (g, 0)),
            pl.BlockSpec((blk, KSEL), lambda g: (g, 0)),
        ],
        out_shape=[
            jax.ShapeDtypeStruct((NB, KSEL), jnp.int32),
            jax.ShapeDtypeStruct((NB, KSEL), jnp.int32),
        ],
    )(rows_a, rows_a, rel_ids, ent_ids)


def _tc_neighbor(rows_c, rows_a, gcn_w, gcn_wb, gcn_b, gate_w, gate_wb,
                 gate_b, blk):
    grid = NB // blk
    self_off = NB * KMAX // blk

    def body(pair_ref, self_ref, gw_ref, gwb_ref, gb_ref, gatew_ref,
             gatewb_ref, gateb_ref, out_ref):
        pairs = pair_ref[...].reshape(blk, KSEL, 2 * EMBED_DIM)
        selfr = self_ref[...]
        proj = lax.dot_general(pairs, gw_ref[...],
                               (((2,), (1,)), ((), ())),
                               preferred_element_type=jnp.float32)
        proj = proj + (gwb_ref[...] + gb_ref[...])[None, None, :]
        proj = jnp.where(proj >= 0, proj, 0.01 * proj)
        agg = jnp.sum(proj, axis=1) / (float(KSEL) + 1e-9)
        lin = jnp.sum(agg * gatew_ref[...], axis=1, keepdims=True)
        gate = jax.nn.sigmoid(lin + (gatewb_ref[0] + gateb_ref[0]))
        final = gate * agg + (1.0 - gate) * selfr
        out_ref[...] = jnp.tanh(final)

    return pl.pallas_call(
        body,
        grid=(grid,),
        in_specs=[
            pl.BlockSpec((blk * 2 * KSEL, EMBED_DIM), lambda g: (g, 0)),
            pl.BlockSpec((blk, EMBED_DIM), lambda g: (self_off + g, 0)),
            pl.BlockSpec((EMBED_DIM, 2 * EMBED_DIM), lambda g: (0, 0)),
            pl.BlockSpec((EMBED_DIM,), lambda g: (0,)),
            pl.BlockSpec((EMBED_DIM,), lambda g: (0,)),
            pl.BlockSpec((1, EMBED_DIM), lambda g: (0, 0)),
            pl.BlockSpec((1,), lambda g: (0,)),
            pl.BlockSpec((1,), lambda g: (0,)),
        ],
        out_specs=pl.BlockSpec((blk, EMBED_DIM), lambda g: (g, 0)),
        out_shape=jax.ShapeDtypeStruct((NB, EMBED_DIM), jnp.float32),
    )(rows_c, rows_a, gcn_w, gcn_wb, gcn_b, gate_w, gate_wb, gate_b)


def _tc_tail(query_vec, support_vec, p1w, p1b, p2w, p2b, lng, lnb, wih, whh,
             bih, bhh, blk):
    d_model = 2 * EMBED_DIM
    grid = B_Q // blk

    def enc(x, p1w, p1b, p2w, p2b, lng, lnb):
        out = jax.nn.relu(_mm_t(x, p1w) + p1b[None, :])
        out = _mm_t(out, p2w) + p2b[None, :]
        y = out + x
        mu = jnp.mean(y, axis=-1, keepdims=True)
        var = jnp.mean((y - mu) ** 2, axis=-1, keepdims=True)
        return lng[None, :] * (y - mu) * lax.rsqrt(var + 1e-6) + lnb[None, :]

    def body(q_ref, sv_ref, p1w_ref, p1b_ref, p2w_ref, p2b_ref, lng_ref,
             lnb_ref, wih_ref, whh_ref, bih_ref, bhh_ref, out_ref):
        p1w, p1b = p1w_ref[...], p1b_ref[...]
        p2w, p2b = p2w_ref[...], p2b_ref[...]
        lng, lnb = lng_ref[...], lnb_ref[...]
        wih, whh = wih_ref[...], whh_ref[...]
        bias = (bih_ref[...] + bhh_ref[...])[None, :]

        sg = jnp.mean(enc(sv_ref[...], p1w, p1b, p2w, p2b, lng, lnb),
                      axis=0, keepdims=True)
        qe = enc(q_ref[...], p1w, p1b, p2w, p2b, lng, lnb)

        qc = _mm_t(qe, wih) + bias
        whh_l = whh[:, :d_model]
        whh_r = whh[:, d_model:]
        rcon = _mm_t(sg, whh_r)

        hid = 2 * d_model
        c = jnp.zeros((blk, hid), jnp.float32)
        h = qe
        for step in range(4):
            if step == 0:
                gates = qc
            else:
                gates = qc + _mm_t(h, whh_l) + rcon
            gi = gates[:, 0 * hid:1 * hid]
            gf = gates[:, 1 * hid:2 * hid]
            gg = gates[:, 2 * hid:3 * hid]
            go = gates[:, 3 * hid:4 * hid]
            c = jax.nn.sigmoid(gf) * c + jax.nn.sigmoid(gi) * jnp.tanh(gg)
            h_r = jax.nn.sigmoid(go) * jnp.tanh(c)
            h = qe + h_r[:, :d_model]
        out_ref[...] = jnp.sum(h * sg, axis=1)

    return pl.pallas_call(
        body,
        grid=(grid,),
        in_specs=[
            pl.BlockSpec((blk, d_model), lambda g: (g, 0)),
            pl.BlockSpec((B_S, d_model), lambda g: (0, 0)),
            pl.BlockSpec((2 * d_model, d_model), lambda g: (0, 0)),
            pl.BlockSpec((2 * d_model,), lambda g: (0,)),
            pl.BlockSpec((d_model, 2 * d_model), lambda g: (0, 0)),
            pl.BlockSpec((d_model,), lambda g: (0,)),
            pl.BlockSpec((d_model,), lambda g: (0,)),
            pl.BlockSpec((d_model,), lambda g: (0,)),
            pl.BlockSpec((8 * d_model, d_model), lambda g: (0, 0)),
            pl.BlockSpec((8 * d_model, 2 * d_model), lambda g: (0, 0)),
            pl.BlockSpec((8 * d_model,), lambda g: (0,)),
            pl.BlockSpec((8 * d_model,), lambda g: (0,)),
        ],
        out_specs=pl.BlockSpec((blk,), lambda g: (g,)),
        out_shape=jax.ShapeDtypeStruct((B_Q,), jnp.float32),
    )(query_vec, support_vec, p1w, p1b, p2w, p2b, lng, lnb, wih, whh, bih,
      bhh)


def kernel(symbol_emb, gcn_w, gcn_wb, gcn_b, gate_w, gate_wb, gate_b, p1w,
           p1b, p2w, p2b, lng, lnb, wih, whh, bih, bhh, query, support, q_l1,
           q_e2, q_deg_l, q_r1, q_e5, q_deg_r, s_l1, s_e2, s_deg_l, s_r1,
           s_e5, s_deg_r):
    conn = jnp.concatenate([q_l1, q_r1, s_l1, s_r1], axis=0)
    conn = jnp.pad(conn, ((0, NB - NROWS), (0, 0), (0, 0)),
                   constant_values=PAD)
    selves = jnp.concatenate([query[:, 0], query[:, 1], support[:, 0],
                              support[:, 1]], axis=0)
    selves = jnp.pad(selves, (0, NB - NROWS), constant_values=PAD)
    rel_ids = conn[:, :, 0]
    ent_ids = conn[:, :, 1]

    ids_a = jnp.concatenate([ent_ids.reshape(-1), selves])
    pad_a = NW * 114 * CHUNK - ids_a.shape[0]
    ids_a = jnp.pad(ids_a, (0, pad_a)).reshape(NW, 114, CHUNK)
    rows_a = _sc_gather(symbol_emb, ids_a, nch=114)

    rel_sel, ent_sel = _tc_simtopk(rows_a, rel_ids, ent_ids, blk=64)

    ids_c = jnp.stack([rel_sel, ent_sel], axis=-1).reshape(-1)
    pad_c = NW * 12 * CHUNK - ids_c.shape[0]
    ids_c = jnp.pad(ids_c, (0, pad_c)).reshape(NW, 12, CHUNK)
    rows_c = _sc_gather(symbol_emb, ids_c, nch=12)

    nbout = _tc_neighbor(rows_c, rows_a, gcn_w, gcn_wb, gcn_b, gate_w,
                         gate_wb, gate_b, blk=64)

    query_vec = jnp.concatenate([nbout[:B_Q], nbout[B_Q:2 * B_Q]], axis=1)
    support_vec = jnp.concatenate(
        [nbout[2 * B_Q:2 * B_Q + B_S], nbout[2 * B_Q + B_S:NROWS]], axis=1)

    return _tc_tail(query_vec, support_vec, p1w, p1b, p2w, p2b, lng, lnb,
                    wih, whh, bih, bhh, blk=256)


# speedup vs baseline: 5.5603x; 3.9406x over previous
import functools

import jax
import jax.numpy as jnp
from jax import lax
from jax.experimental import pallas as pl
from jax.experimental.pallas import tpu as pltpu
from jax.experimental.pallas import tpu_sc as plsc

EMBED_DIM = 128
NUM_SYM = 100000
PAD = NUM_SYM
KMAX = 200
KSEL = 10
B_Q, B_S = 1024, 5
NROWS = 2 * B_Q + 2 * B_S
NB = 2304
NW = 32
CHUNK = 128


def _mm_t(a, b):
    return lax.dot_general(a, b, (((1,), (1,)), ((), ())),
                           preferred_element_type=jnp.float32)


def _sc_gather(table, ids_2d, nch):
    n_out = NW * nch * CHUNK
    per_w = nch * CHUNK
    mesh = plsc.VectorSubcoreMesh(core_axis_name="c", subcore_axis_name="s")

    @functools.partial(
        pl.kernel, mesh=mesh,
        out_type=jax.ShapeDtypeStruct((n_out, EMBED_DIM), jnp.float32),
        scratch_types=[
            pltpu.VMEM((nch, CHUNK), jnp.int32),
            pltpu.VMEM((CHUNK, EMBED_DIM), jnp.float32),
            pltpu.VMEM((CHUNK, EMBED_DIM), jnp.float32),
            pltpu.SemaphoreType.DMA,
            pltpu.SemaphoreType.DMA,
        ],
    )
    def k(table_hbm, ids_hbm, out_hbm, idx_v, buf0, buf1, sem0, sem1):
        wid = lax.axis_index("s") * 2 + lax.axis_index("c")
        pltpu.sync_copy(ids_hbm.at[wid], idx_v)
        bufs = (buf0, buf1)
        sems = (sem0, sem1)

        def start(ci, b):
            pltpu.make_async_copy(
                table_hbm.at[idx_v.at[ci]], bufs[b], sems[b]).start()

        def drain(ci, b):
            pltpu.make_async_copy(
                table_hbm.at[idx_v.at[ci]], bufs[b], sems[b]).wait()
            off = wid * per_w + ci * CHUNK
            pltpu.sync_copy(bufs[b], out_hbm.at[pl.ds(off, CHUNK)])

        start(0, 0)

        def body(i, carry):
            ci0 = 2 * i
            ci1 = ci0 + 1

            @pl.when(ci1 < nch)
            def _():
                start(ci1, 1)

            drain(ci0, 0)

            @pl.when(ci0 + 2 < nch)
            def _():
                start(ci0 + 2, 0)

            @pl.when(ci1 < nch)
            def _():
                drain(ci1, 1)

            return carry

        lax.fori_loop(0, (nch + 1) // 2, body, 0)

    return k(table, ids_2d)


def _tc_simtopk(rows_a, rel_ids, ent_ids, blk):
    grid = NB // blk
    self_off = NB * KMAX // blk

    def body(ent_ref, self_ref, rel_ref, eid_ref, relo_ref, ento_ref):
        ent = ent_ref[...].reshape(blk, KMAX, EMBED_DIM)
        selfr = self_ref[...]
        rels = rel_ref[...]
        eids = eid_ref[...]

        sn = selfr * lax.rsqrt(jnp.maximum(
            jnp.sum(selfr * selfr, axis=-1, keepdims=True), 1e-24))
        en = ent * lax.rsqrt(jnp.maximum(
            jnp.sum(ent * ent, axis=-1, keepdims=True), 1e-24))
        sim = jnp.sum(sn[:, None, :] * en, axis=-1)
        sim = sim - jnp.where(rels == PAD, 1e9, 0.0).astype(jnp.float32)

        iota = lax.broadcasted_iota(jnp.int32, (blk, KMAX), 1)
        relf = rels.astype(jnp.float32)
        entf = eids.astype(jnp.float32)
        rel_cols, ent_cols = [], []
        for _ in range(KSEL):
            m = jnp.max(sim, axis=1, keepdims=True)
            ismax = sim == m
            idxsel = jnp.min(jnp.where(ismax, iota, KMAX + 1), axis=1,
                             keepdims=True)
            chosen = iota == idxsel
            rel_cols.append(jnp.sum(jnp.where(chosen, relf, 0.0), axis=1,
                                    keepdims=True))
            ent_cols.append(jnp.sum(jnp.where(chosen, entf, 0.0), axis=1,
                                    keepdims=True))
            sim = jnp.where(chosen, -1e38, sim)
        relo_ref[...] = jnp.concatenate(rel_cols, axis=1).astype(jnp.int32)
        ento_ref[...] = jnp.concatenate(ent_cols, axis=1).astype(jnp.int32)

    return pl.pallas_call(
        body,
        grid=(grid,),
        in_specs=[
            pl.BlockSpec((blk * KMAX, EMBED_DIM), lambda g: (g, 0)),
            pl.BlockSpec((blk, EMBED_DIM), lambda g: (self_off + g, 0)),
            pl.BlockSpec((blk, KMAX), lambda g: (g, 0)),
            pl.BlockSpec((blk, KMAX), lambda g: (g, 0)),
        ],
        out_specs=[
            pl.BlockSpec((blk, KSEL), lambda g: (g, 0)),
            pl.BlockSpec((blk, KSEL), lambda g: (g, 0)),
        ],
        out_shape=[
            jax.ShapeDtypeStruct((NB, KSEL), jnp.int32),
            jax.ShapeDtypeStruct((NB, KSEL), jnp.int32),
        ],
    )(rows_a, rows_a, rel_ids, ent_ids)


def _tc_neighbor(rows_c, rows_a, gcn_w, gcn_wb, gcn_b, gate_w, gate_wb,
                 gate_b, blk):
    grid = NB // blk
    self_off = NB * KMAX // blk

    def body(pair_ref, self_ref, gw_ref, gwb_ref, gb_ref, gatew_ref,
             gatewb_ref, gateb_ref, out_ref):
        pairs = pair_ref[...].reshape(blk, KSEL, 2 * EMBED_DIM)
        selfr = self_ref[...]
        proj = lax.dot_general(pairs, gw_ref[...],
                               (((2,), (1,)), ((), ())),
                               preferred_element_type=jnp.float32)
        proj = proj + (gwb_ref[...] + gb_ref[...])[None, None, :]
        proj = jnp.where(proj >= 0, proj, 0.01 * proj)
        agg = jnp.sum(proj, axis=1) / (float(KSEL) + 1e-9)
        lin = jnp.sum(agg * gatew_ref[...], axis=1, keepdims=True)
        gate = jax.nn.sigmoid(lin + (gatewb_ref[0] + gateb_ref[0]))
        final = gate * agg + (1.0 - gate) * selfr
        out_ref[...] = jnp.tanh(final)

    return pl.pallas_call(
        body,
        grid=(grid,),
        in_specs=[
            pl.BlockSpec((blk * 2 * KSEL, EMBED_DIM), lambda g: (g, 0)),
            pl.BlockSpec((blk, EMBED_DIM), lambda g: (self_off + g, 0)),
            pl.BlockSpec((EMBED_DIM, 2 * EMBED_DIM), lambda g: (0, 0)),
            pl.BlockSpec((EMBED_DIM,), lambda g: (0,)),
            pl.BlockSpec((EMBED_DIM,), lambda g: (0,)),
            pl.BlockSpec((1, EMBED_DIM), lambda g: (0, 0)),
            pl.BlockSpec((1,), lambda g: (0,)),
            pl.BlockSpec((1,), lambda g: (0,)),
        ],
        out_specs=pl.BlockSpec((blk, EMBED_DIM), lambda g: (g, 0)),
        out_shape=jax.ShapeDtypeStruct((NB, EMBED_DIM), jnp.float32),
    )(rows_c, rows_a, gcn_w, gcn_wb, gcn_b, gate_w, gate_wb, gate_b)


def _tc_tail(query_vec, support_vec, p1w, p1b, p2w, p2b, lng, lnb, wih, whh,
             bih, bhh, blk):
    d_model = 2 * EMBED_DIM
    grid = B_Q // blk

    def enc(x, p1w, p1b, p2w, p2b, lng, lnb):
        out = jax.nn.relu(_mm_t(x, p1w) + p1b[None, :])
        out = _mm_t(out, p2w) + p2b[None, :]
        y = out + x
        mu = jnp.mean(y, axis=-1, keepdims=True)
        var = jnp.mean((y - mu) ** 2, axis=-1, keepdims=True)
        return lng[None, :] * (y - mu) * lax.rsqrt(var + 1e-6) + lnb[None, :]

    def body(q_ref, sv_ref, p1w_ref, p1b_ref, p2w_ref, p2b_ref, lng_ref,
             lnb_ref, wih_ref, whh_ref, bih_ref, bhh_ref, out_ref):
        p1w, p1b = p1w_ref[...], p1b_ref[...]
        p2w, p2b = p2w_ref[...], p2b_ref[...]
        lng, lnb = lng_ref[...], lnb_ref[...]
        wih, whh = wih_ref[...], whh_ref[...]
        bias = (bih_ref[...] + bhh_ref[...])[None, :]

        sg = jnp.mean(enc(sv_ref[...], p1w, p1b, p2w, p2b, lng, lnb),
                      axis=0, keepdims=True)
        qe = enc(q_ref[...], p1w, p1b, p2w, p2b, lng, lnb)

        qc = _mm_t(qe, wih) + bias
        whh_l = whh[:, :d_model]
        whh_r = whh[:, d_model:]
        rcon = _mm_t(sg, whh_r)

        hid = 2 * d_model
        c = jnp.zeros((blk, hid), jnp.float32)
        h = qe
        for step in range(4):
            if step == 0:
                gates = qc
            else:
                gates = qc + _mm_t(h, whh_l) + rcon
            gi = gates[:, 0 * hid:1 * hid]
            gf = gates[:, 1 * hid:2 * hid]
            gg = gates[:, 2 * hid:3 * hid]
            go = gates[:, 3 * hid:4 * hid]
            c = jax.nn.sigmoid(gf) * c + jax.nn.sigmoid(gi) * jnp.tanh(gg)
            h_r = jax.nn.sigmoid(go) * jnp.tanh(c)
            h = qe + h_r[:, :d_model]
        out_ref[...] = jnp.sum(h * sg, axis=1)

    return pl.pallas_call(
        body,
        grid=(grid,),
        in_specs=[
            pl.BlockSpec((blk, d_model), lambda g: (g, 0)),
            pl.BlockSpec((B_S, d_model), lambda g: (0, 0)),
            pl.BlockSpec((2 * d_model, d_model), lambda g: (0, 0)),
            pl.BlockSpec((2 * d_model,), lambda g: (0,)),
            pl.BlockSpec((d_model, 2 * d_model), lambda g: (0, 0)),
            pl.BlockSpec((d_model,), lambda g: (0,)),
            pl.BlockSpec((d_model,), lambda g: (0,)),
            pl.BlockSpec((d_model,), lambda g: (0,)),
            pl.BlockSpec((8 * d_model, d_model), lambda g: (0, 0)),
            pl.BlockSpec((8 * d_model, 2 * d_model), lambda g: (0, 0)),
            pl.BlockSpec((8 * d_model,), lambda g: (0,)),
            pl.BlockSpec((8 * d_model,), lambda g: (0,)),
        ],
        out_specs=pl.BlockSpec((blk,), lambda g: (g,)),
        out_shape=jax.ShapeDtypeStruct((B_Q,), jnp.float32),
    )(query_vec, support_vec, p1w, p1b, p2w, p2b, lng, lnb, wih, whh, bih,
      bhh)


def kernel(symbol_emb, gcn_w, gcn_wb, gcn_b, gate_w, gate_wb, gate_b, p1w,
           p1b, p2w, p2b, lng, lnb, wih, whh, bih, bhh, query, support, q_l1,
           q_e2, q_deg_l, q_r1, q_e5, q_deg_r, s_l1, s_e2, s_deg_l, s_r1,
           s_e5, s_deg_r):
    npad = NB - NROWS
    conn = jnp.concatenate([q_l1, q_r1, s_l1, s_r1], axis=0)
    conn_fill = (jnp.arange(npad * KMAX * 2, dtype=jnp.int32) % NUM_SYM
                 ).reshape(npad, KMAX, 2)
    conn = jnp.concatenate([conn, conn_fill], axis=0)
    selves = jnp.concatenate([query[:, 0], query[:, 1], support[:, 0],
                              support[:, 1],
                              jnp.arange(npad, dtype=jnp.int32) % NUM_SYM])
    rel_ids = conn[:, :, 0]
    ent_ids = conn[:, :, 1]

    ids_a = jnp.concatenate([ent_ids.reshape(-1), selves])
    pad_a = NW * 114 * CHUNK - ids_a.shape[0]
    ids_a = jnp.concatenate(
        [ids_a, jnp.arange(pad_a, dtype=jnp.int32) % NUM_SYM]
    ).reshape(NW, 114, CHUNK)
    rows_a = _sc_gather(symbol_emb, ids_a, nch=114)

    rel_sel, ent_sel = _tc_simtopk(rows_a, rel_ids, ent_ids, blk=64)

    ids_c = jnp.stack([rel_sel, ent_sel], axis=-1).reshape(-1)
    pad_c = NW * 12 * CHUNK - ids_c.shape[0]
    ids_c = jnp.concatenate(
        [ids_c, jnp.arange(pad_c, dtype=jnp.int32) % NUM_SYM]
    ).reshape(NW, 12, CHUNK)
    rows_c = _sc_gather(symbol_emb, ids_c, nch=12)

    nbout = _tc_neighbor(rows_c, rows_a, gcn_w, gcn_wb, gcn_b, gate_w,
                         gate_wb, gate_b, blk=64)

    query_vec = jnp.concatenate([nbout[:B_Q], nbout[B_Q:2 * B_Q]], axis=1)
    support_vec = jnp.concatenate(
        [nbout[2 * B_Q:2 * B_Q + B_S], nbout[2 * B_Q + B_S:NROWS]], axis=1)

    return _tc_tail(query_vec, support_vec, p1w, p1b, p2w, p2b, lng, lnb,
                    wih, whh, bih, bhh, blk=256)
